# Initial kernel scaffold; baseline (speedup 1.0000x reference)
import jax
import jax.numpy as jnp
from jax.experimental import pallas as pl


def kernel(feat, edge_index, review_feat, edge_weight, W_node, W_review):
    raise NotImplementedError("write your pallas kernel here")


# speedup vs baseline: 3.5284x; 3.5284x over previous
import functools

import jax
import jax.numpy as jnp
from jax import lax
from jax.experimental import pallas as pl
from jax.experimental.pallas import tpu as pltpu
from jax.experimental.pallas import tpu_sc as plsc

N_NODES = 100000
N_EDGES = 1600000
G = 128
CH = 8
TILES = 16
GP_T = 784
NG_REAL = N_EDGES // G
NG_PAD = TILES * GP_T
E_PAD = NG_PAD * G
NCHUNK = GP_T // CH
ROWS_T = N_NODES // TILES
ZROWS = 625


def _sc_segment_sums(feat0, feat1, review, srcp, dstp, wp):
    mesh = plsc.VectorSubcoreMesh(core_axis_name="c", subcore_axis_name="s")

    @functools.partial(
        pl.kernel,
        out_type=jax.ShapeDtypeStruct((N_NODES, 64), jnp.float32),
        mesh=mesh,
        scratch_types=[
            pltpu.VMEM_SHARED((N_NODES, 16), jnp.float32),
            pltpu.VMEM((CH, G), jnp.int32),
            pltpu.VMEM((CH, G), jnp.int32),
            pltpu.VMEM((CH, G), jnp.float32),
            pltpu.VMEM((CH * G, 16), jnp.float32),
            pltpu.VMEM((ZROWS, 16), jnp.float32),
            pltpu.SemaphoreType.DMA,
        ],
        compiler_params=pltpu.CompilerParams(use_tc_tiling_on_sc=False),
    )
    def k(feat0_h, feat1_h, review_h, srcp_h, dstp_h, wp_h, out_h,
          acc, src_v, dst_v, w_v, rows_v, zbuf, gsem):
        c = lax.axis_index("c")
        s = lax.axis_index("s")
        r0 = s * ROWS_T

        @pl.loop(0, ZROWS)
        def _zb(i):
            zbuf[i, :] = jnp.zeros((16,), jnp.float32)

        for phase in range(2):
            @pl.loop(0, ROWS_T // ZROWS)
            def _z(kk):
                pltpu.sync_copy(zbuf, acc.at[pl.ds(r0 + kk * ZROWS, ZROWS)])

            plsc.subcore_barrier()

            g0 = s * GP_T

            @pl.loop(0, NCHUNK)
            def _chunk(i):
                gbase = g0 + i * CH
                pltpu.sync_copy(dstp_h.at[pl.ds(gbase, CH)], dst_v)
                pltpu.sync_copy(wp_h.at[pl.ds(gbase, CH)], w_v)
                if phase == 0:
                    pltpu.sync_copy(srcp_h.at[pl.ds(gbase, CH)], src_v)

                    @pl.when(c == 0)
                    def _g0():
                        ds_ = [pltpu.async_copy(feat0_h.at[src_v.at[j]],
                                                rows_v.at[pl.ds(j * G, G)], gsem)
                               for j in range(CH)]
                        for d in ds_:
                            d.wait()

                    @pl.when(c == 1)
                    def _g1():
                        ds_ = [pltpu.async_copy(feat1_h.at[src_v.at[j]],
                                                rows_v.at[pl.ds(j * G, G)], gsem)
                               for j in range(CH)]
                        for d in ds_:
                            d.wait()
                else:
                    for j in range(CH):
                        gj = jnp.minimum(gbase + j, NG_REAL - 1)
                        pltpu.sync_copy(
                            review_h.at[pl.ds(gj * G, G), pl.ds(c * 16, 16)],
                            rows_v.at[pl.ds(j * G, G)])

                for j in range(CH):
                    @plsc.parallel_loop(0, G // 16, unroll=2)
                    def _m(kk):
                        w16 = w_v[j, pl.ds(kk * 16, 16)]
                        for t in range(16):
                            r = j * G + kk * 16 + t
                            rows_v[r, :] = rows_v[r, :] * w16[t]

                for j in range(CH):
                    pltpu.sync_copy(rows_v.at[pl.ds(j * G, G)],
                                    acc.at[dst_v.at[j]], add=True)

            plsc.subcore_barrier()
            colbase = phase * 32 + c * 16
            pltpu.sync_copy(acc.at[pl.ds(r0, ROWS_T)],
                            out_h.at[pl.ds(r0, ROWS_T), pl.ds(colbase, 16)])

    return k(feat0, feat1, review, srcp, dstp, wp)


def _tc_matmul(x, wcat):
    BR = 2000

    def body(x_ref, w_ref, o_ref):
        o_ref[...] = jnp.dot(x_ref[...], w_ref[...],
                             preferred_element_type=jnp.float32)

    return pl.pallas_call(
        body,
        grid=(N_NODES // BR,),
        in_specs=[pl.BlockSpec((BR, 64), lambda i: (i, 0)),
                  pl.BlockSpec((64, 32), lambda i: (0, 0))],
        out_specs=pl.BlockSpec((BR, 32), lambda i: (i, 0)),
        out_shape=jax.ShapeDtypeStruct((N_NODES, 32), jnp.float32),
    )(x, wcat)


def kernel(feat, edge_index, review_feat, edge_weight, W_node, W_review):
    src = edge_index[0].astype(jnp.int32)
    dst = edge_index[1].astype(jnp.int32)
    w = edge_weight.reshape(-1)
    pad = E_PAD - N_EDGES
    srcp = jnp.concatenate([src, jnp.zeros((pad,), src.dtype)]).reshape(NG_PAD, G)
    dstp = jnp.concatenate([dst, jnp.zeros((pad,), dst.dtype)]).reshape(NG_PAD, G)
    wp = jnp.concatenate([w, jnp.zeros((pad,), w.dtype)]).reshape(NG_PAD, G)
    feat0 = feat[:, :16]
    feat1 = feat[:, 16:]
    out64 = _sc_segment_sums(feat0, feat1, review_feat, srcp, dstp, wp)
    wcat = jnp.concatenate([W_node.T, W_review.T], axis=0)
    return _tc_matmul(out64, wcat)


# speedup vs baseline: 4.9374x; 1.3993x over previous
import functools

import jax
import jax.numpy as jnp
from jax import lax
from jax.experimental import pallas as pl
from jax.experimental.pallas import tpu as pltpu
from jax.experimental.pallas import tpu_sc as plsc

N_NODES = 100000
N_EDGES = 1600000
G = 128
CH = 4
TILES = 16
NG = N_EDGES // G
GP_T = NG // TILES
REM = NG - GP_T * TILES
FULL = GP_T // CH
TAIL_BASE = FULL * CH
ROWS_T = N_NODES // TILES
ZROWS = 125


def _make_sc_kernel():
    mesh = plsc.VectorSubcoreMesh(core_axis_name="c", subcore_axis_name="s")

    @functools.partial(
        pl.kernel,
        out_type=jax.ShapeDtypeStruct((N_NODES, 64), jnp.float32),
        mesh=mesh,
        scratch_types=[
            pltpu.VMEM_SHARED((N_NODES, 16), jnp.float32),
            pltpu.VMEM((2, CH, G), jnp.int32),
            pltpu.VMEM((3, CH, G), jnp.int32),
            pltpu.VMEM((2, CH, G), jnp.float32),
            pltpu.VMEM((3, CH, G, 16), jnp.float32),
            pltpu.VMEM((ZROWS, 16), jnp.float32),
            pltpu.SemaphoreType.DMA,
            pltpu.SemaphoreType.DMA,
            pltpu.SemaphoreType.DMA((3,)),
        ],
        compiler_params=pltpu.CompilerParams(use_tc_tiling_on_sc=False),
    )
    def k(feat0_h, feat1_h, ei_h, w_h, rv_h, out_h,
          acc, src_v, dst_v, w_v, half_v, zbuf,
          sem_in, sem_g, sem_s):
        c = lax.axis_index("c")
        s = lax.axis_index("s")
        r0 = s * ROWS_T
        coff = c * 16
        base_g = s * GP_T + jnp.minimum(s, REM)
        tail = GP_T + jnp.where(s < REM, 1, 0) - TAIL_BASE

        @pl.loop(0, ZROWS)
        def _zb(i):
            zbuf[i, :] = jnp.zeros((16,), jnp.float32)

        def in_descs(phase, i, b2, b3, make):
            gb = base_g + i * CH
            op = pltpu.make_async_copy if make else pltpu.async_copy
            ds_ = [op(ei_h.at[1, pl.ds(gb, CH)], dst_v.at[b3], sem_in),
                   op(w_h.at[pl.ds(gb, CH)], w_v.at[b2], sem_in)]
            if phase == 0:
                ds_.append(op(ei_h.at[0, pl.ds(gb, CH)], src_v.at[b2], sem_in))
            return ds_

        def fire_data(phase, i, b2, b3):
            if phase == 0:
                @pl.when(c == 0)
                def _f0():
                    for j in range(CH):
                        pltpu.async_copy(feat0_h.at[src_v.at[b2, j]],
                                         half_v.at[b3, j], sem_g)

                @pl.when(c == 1)
                def _f1():
                    for j in range(CH):
                        pltpu.async_copy(feat1_h.at[src_v.at[b2, j]],
                                         half_v.at[b3, j], sem_g)
            else:
                gb = base_g + i * CH
                pltpu.async_copy(rv_h.at[pl.ds(gb, CH), :, pl.ds(coff, 16)],
                                 half_v.at[b3], sem_g)

        def drain_data(phase, i, b2, b3):
            if phase == 0:
                for j in range(CH):
                    pltpu.make_async_copy(feat0_h.at[src_v.at[b2, j]],
                                          half_v.at[b3, j], sem_g).wait()
            else:
                gb = base_g + i * CH
                pltpu.make_async_copy(
                    rv_h.at[pl.ds(gb, CH), :, pl.ds(coff, 16)],
                    half_v.at[b3], sem_g).wait()

        def compute(b2, b3, nj=CH):
            for j in range(nj):
                @plsc.parallel_loop(0, G // 16, unroll=2)
                def _m(kk):
                    w16 = w_v[b2, j, pl.ds(kk * 16, 16)]
                    for t in range(16):
                        e = kk * 16 + t
                        half_v[b3, j, e, :] = half_v[b3, j, e, :] * w16[t]

        def fire_scatter(b3):
            for j in range(CH):
                pltpu.async_copy(half_v.at[b3, j], acc.at[dst_v.at[b3, j]],
                                 sem_s.at[b3], add=True)

        def drain_scatter(b3):
            for j in range(CH):
                pltpu.make_async_copy(half_v.at[b3, j], acc.at[dst_v.at[b3, j]],
                                      sem_s.at[b3]).wait()

        for phase in range(2):
            @pl.loop(0, ROWS_T // ZROWS)
            def _z(kk):
                pltpu.sync_copy(zbuf, acc.at[pl.ds(r0 + kk * ZROWS, ZROWS)])

            plsc.subcore_barrier()

            for d in in_descs(phase, 0, 0, 0, make=False):
                d.wait()
            fire_data(phase, 0, 0, 0)

            @pl.loop(0, FULL)
            def _chunk(i):
                b2 = lax.rem(i, 2)
                nb2 = 1 - b2
                b3 = lax.rem(i, 3)
                nb3 = lax.rem(i + 1, 3)

                @pl.when(i >= 2)
                def _dsc():
                    drain_scatter(nb3)

                @pl.when(i < FULL - 1)
                def _pf():
                    in_descs(phase, i + 1, nb2, nb3, make=False)

                drain_data(phase, i, b2, b3)
                compute(b2, b3)
                fire_scatter(b3)

                @pl.when(i < FULL - 1)
                def _ng():
                    for d in in_descs(phase, i + 1, nb2, nb3, make=True):
                        d.wait()
                    fire_data(phase, i + 1, nb2, nb3)

            drain_scatter((FULL - 2) % 3)
            drain_scatter((FULL - 1) % 3)

            @pl.loop(0, tail)
            def _tail(tg):
                g = base_g + TAIL_BASE + tg
                pltpu.sync_copy(ei_h.at[1, g], dst_v.at[0, 0])
                pltpu.sync_copy(w_h.at[g], w_v.at[0, 0])
                if phase == 0:
                    pltpu.sync_copy(ei_h.at[0, g], src_v.at[0, 0])

                    @pl.when(c == 0)
                    def _t0():
                        pltpu.async_copy(feat0_h.at[src_v.at[0, 0]],
                                         half_v.at[0, 0], sem_g).wait()

                    @pl.when(c == 1)
                    def _t1():
                        pltpu.async_copy(feat1_h.at[src_v.at[0, 0]],
                                         half_v.at[0, 0], sem_g).wait()
                else:
                    pltpu.sync_copy(rv_h.at[g, :, pl.ds(coff, 16)],
                                    half_v.at[0, 0])
                compute(0, 0, nj=1)
                pltpu.sync_copy(half_v.at[0, 0], acc.at[dst_v.at[0, 0]],
                                add=True)

            plsc.subcore_barrier()
            colbase = phase * 32 + coff
            pltpu.sync_copy(acc.at[pl.ds(r0, ROWS_T)],
                            out_h.at[pl.ds(r0, ROWS_T), pl.ds(colbase, 16)])
            if phase == 0:
                plsc.subcore_barrier()

    return k


def _tc_matmul(x, wcat):
    BR = 2000

    def body(x_ref, w_ref, o_ref):
        o_ref[...] = jnp.dot(x_ref[...], w_ref[...],
                             preferred_element_type=jnp.float32)

    return pl.pallas_call(
        body,
        grid=(N_NODES // BR,),
        in_specs=[pl.BlockSpec((BR, 64), lambda i: (i, 0)),
                  pl.BlockSpec((64, 32), lambda i: (0, 0))],
        out_specs=pl.BlockSpec((BR, 32), lambda i: (i, 0)),
        out_shape=jax.ShapeDtypeStruct((N_NODES, 32), jnp.float32),
    )(x, wcat)


def kernel(feat, edge_index, review_feat, edge_weight, W_node, W_review):
    ei3 = edge_index.astype(jnp.int32).reshape(2, NG, G)
    w3 = edge_weight.reshape(NG, G)
    rv3 = review_feat.reshape(NG, G, 32)
    feat0 = feat[:, :16]
    feat1 = feat[:, 16:]
    out64 = _make_sc_kernel()(feat0, feat1, ei3, w3, rv3)
    wcat = jnp.concatenate([W_node.T, W_review.T], axis=0)
    return _tc_matmul(out64, wcat)


# speedup vs baseline: 6.0736x; 1.2301x over previous
import functools

import jax
import jax.numpy as jnp
from jax import lax
from jax.experimental import pallas as pl
from jax.experimental.pallas import tpu as pltpu
from jax.experimental.pallas import tpu_sc as plsc

N_NODES = 100000
N_EDGES = 1600000
G = 128
CH = 4
TILES = 16
NG = N_EDGES // G
GP_T = NG // TILES
REM = NG - GP_T * TILES
FULL = GP_T // CH
TAIL_BASE = FULL * CH
ROWS_T = N_NODES // TILES
ZROWS = 125


def _sc_body(phase, refs):
    if phase == 0:
        (feat0_h, feat1_h, src_h, dst_h, w_h, out_h,
         acc, src_v, dst_v, w_v, half_v, zbuf, sem_in, sem_g, sem_s) = refs
    else:
        (rv_h, dst_h, w_h, out_h,
         acc, src_v, dst_v, w_v, half_v, zbuf, sem_in, sem_g, sem_s) = refs

    c = lax.axis_index("c")
    s = lax.axis_index("s")
    r0 = s * ROWS_T
    coff = c * 16
    base_g = s * GP_T + jnp.minimum(s, REM)
    tail = GP_T + jnp.where(s < REM, 1, 0) - TAIL_BASE

    @pl.loop(0, ZROWS)
    def _zb(i):
        zbuf[i, :] = jnp.zeros((16,), jnp.float32)

    def in_descs(i, b2, b3, make):
        gb = base_g + i * CH
        op = pltpu.make_async_copy if make else pltpu.async_copy
        ds_ = []
        for j in range(CH):
            e0 = (gb + j) * G
            ds_.append(op(dst_h.at[pl.ds(e0, G)], dst_v.at[b3, j], sem_in))
            ds_.append(op(w_h.at[pl.ds(e0, G)], w_v.at[b2, j], sem_in))
            if phase == 0:
                ds_.append(op(src_h.at[pl.ds(e0, G)], src_v.at[b2, j], sem_in))
        return ds_

    def fire_data(i, b2, b3):
        if phase == 0:
            @pl.when(c == 0)
            def _f0():
                for j in range(CH):
                    pltpu.async_copy(feat0_h.at[src_v.at[b2, j]],
                                     half_v.at[b3, j], sem_g)

            @pl.when(c == 1)
            def _f1():
                for j in range(CH):
                    pltpu.async_copy(feat1_h.at[src_v.at[b2, j]],
                                     half_v.at[b3, j], sem_g)
        else:
            gb = base_g + i * CH
            pltpu.async_copy(rv_h.at[pl.ds(gb, CH), :, pl.ds(coff, 16)],
                             half_v.at[b3], sem_g)

    def drain_data(i, b2, b3):
        if phase == 0:
            for j in range(CH):
                pltpu.make_async_copy(feat0_h.at[src_v.at[b2, j]],
                                      half_v.at[b3, j], sem_g).wait()
        else:
            gb = base_g + i * CH
            pltpu.make_async_copy(
                rv_h.at[pl.ds(gb, CH), :, pl.ds(coff, 16)],
                half_v.at[b3], sem_g).wait()

    def compute(b2, b3, nj=CH):
        for j in range(nj):
            @plsc.parallel_loop(0, G // 16, unroll=2)
            def _m(kk):
                w16 = w_v[b2, j, pl.ds(kk * 16, 16)]
                for t in range(16):
                    e = kk * 16 + t
                    half_v[b3, j, e, :] = half_v[b3, j, e, :] * w16[t]

    def fire_scatter(b3):
        for j in range(CH):
            pltpu.async_copy(half_v.at[b3, j], acc.at[dst_v.at[b3, j]],
                             sem_s.at[b3], add=True)

    def drain_scatter(b3):
        for j in range(CH):
            pltpu.make_async_copy(half_v.at[b3, j], acc.at[dst_v.at[b3, j]],
                                  sem_s.at[b3]).wait()

    @pl.loop(0, ROWS_T // ZROWS)
    def _z(kk):
        pltpu.sync_copy(zbuf, acc.at[pl.ds(r0 + kk * ZROWS, ZROWS)])

    plsc.subcore_barrier()

    for d in in_descs(0, 0, 0, make=False):
        d.wait()
    fire_data(0, 0, 0)

    @pl.loop(0, FULL)
    def _chunk(i):
        b2 = lax.rem(i, 2)
        nb2 = 1 - b2
        b3 = lax.rem(i, 3)
        nb3 = lax.rem(i + 1, 3)

        @pl.when(i >= 2)
        def _dsc():
            drain_scatter(nb3)

        @pl.when(i < FULL - 1)
        def _pf():
            in_descs(i + 1, nb2, nb3, make=False)

        drain_data(i, b2, b3)
        compute(b2, b3)
        fire_scatter(b3)

        @pl.when(i < FULL - 1)
        def _ng():
            for d in in_descs(i + 1, nb2, nb3, make=True):
                d.wait()
            fire_data(i + 1, nb2, nb3)

    drain_scatter((FULL - 2) % 3)
    drain_scatter((FULL - 1) % 3)

    @pl.loop(0, tail)
    def _tail(tg):
        g = base_g + TAIL_BASE + tg
        e0 = g * G
        pltpu.sync_copy(dst_h.at[pl.ds(e0, G)], dst_v.at[0, 0])
        pltpu.sync_copy(w_h.at[pl.ds(e0, G)], w_v.at[0, 0])
        if phase == 0:
            pltpu.sync_copy(src_h.at[pl.ds(e0, G)], src_v.at[0, 0])

            @pl.when(c == 0)
            def _t0():
                pltpu.async_copy(feat0_h.at[src_v.at[0, 0]],
                                 half_v.at[0, 0], sem_g).wait()

            @pl.when(c == 1)
            def _t1():
                pltpu.async_copy(feat1_h.at[src_v.at[0, 0]],
                                 half_v.at[0, 0], sem_g).wait()
        else:
            pltpu.sync_copy(rv_h.at[g, :, pl.ds(coff, 16)], half_v.at[0, 0])
        compute(0, 0, nj=1)
        pltpu.sync_copy(half_v.at[0, 0], acc.at[dst_v.at[0, 0]], add=True)

    plsc.subcore_barrier()
    pltpu.sync_copy(acc.at[pl.ds(r0, ROWS_T)],
                    out_h.at[pl.ds(r0, ROWS_T), pl.ds(coff, 16)])


_SCRATCH = [
    pltpu.VMEM_SHARED((N_NODES, 16), jnp.float32),
    pltpu.VMEM((2, CH, G), jnp.int32),
    pltpu.VMEM((3, CH, G), jnp.int32),
    pltpu.VMEM((2, CH, G), jnp.float32),
    pltpu.VMEM((3, CH, G, 16), jnp.float32),
    pltpu.VMEM((ZROWS, 16), jnp.float32),
    pltpu.SemaphoreType.DMA,
    pltpu.SemaphoreType.DMA,
    pltpu.SemaphoreType.DMA((3,)),
]


def _make_phase_a():
    mesh = plsc.VectorSubcoreMesh(core_axis_name="c", subcore_axis_name="s")

    @functools.partial(
        pl.kernel,
        out_type=jax.ShapeDtypeStruct((N_NODES, 32), jnp.float32),
        mesh=mesh,
        scratch_types=list(_SCRATCH),
        compiler_params=pltpu.CompilerParams(use_tc_tiling_on_sc=False),
    )
    def ka(*refs):
        _sc_body(0, refs)

    return ka


def _make_phase_b():
    mesh = plsc.VectorSubcoreMesh(core_axis_name="c", subcore_axis_name="s")

    @functools.partial(
        pl.kernel,
        out_type=jax.ShapeDtypeStruct((N_NODES, 32), jnp.float32),
        mesh=mesh,
        scratch_types=list(_SCRATCH),
        compiler_params=pltpu.CompilerParams(use_tc_tiling_on_sc=False),
    )
    def kb(*refs):
        _sc_body(1, refs)

    return kb


def _tc_matmul(a, b, wn_t, wr_t):
    BR = 2000

    def body(a_ref, b_ref, wn_ref, wr_ref, o_ref):
        o_ref[...] = (
            jnp.dot(a_ref[...], wn_ref[...], preferred_element_type=jnp.float32)
            + jnp.dot(b_ref[...], wr_ref[...], preferred_element_type=jnp.float32))

    return pl.pallas_call(
        body,
        grid=(N_NODES // BR,),
        in_specs=[pl.BlockSpec((BR, 32), lambda i: (i, 0)),
                  pl.BlockSpec((BR, 32), lambda i: (i, 0)),
                  pl.BlockSpec((32, 32), lambda i: (0, 0)),
                  pl.BlockSpec((32, 32), lambda i: (0, 0))],
        out_specs=pl.BlockSpec((BR, 32), lambda i: (i, 0)),
        out_shape=jax.ShapeDtypeStruct((N_NODES, 32), jnp.float32),
    )(a, b, wn_t, wr_t)


def kernel(feat, edge_index, review_feat, edge_weight, W_node, W_review):
    ei = edge_index.astype(jnp.int32)
    src = ei[0]
    dst = ei[1]
    w = edge_weight.reshape(-1)
    rv3 = review_feat.reshape(NG, G, 32)
    feat0 = feat[:, :16]
    feat1 = feat[:, 16:]
    a64 = _make_phase_a()(feat0, feat1, src, dst, w)
    b64 = _make_phase_b()(rv3, dst, w)
    return _tc_matmul(a64, b64, W_node.T, W_review.T)
